# Initial kernel scaffold; baseline (speedup 1.0000x reference)
import jax
import jax.numpy as jnp
from jax.experimental import pallas as pl


def kernel(hidden, history, W, b):
    raise NotImplementedError("write your pallas kernel here")


# speedup vs baseline: 1.4563x; 1.4563x over previous
import math

import jax
import jax.numpy as jnp
from jax.experimental import pallas as pl

B, T, D = 4, 8192, 1024
PRIOR_WEIGHT = 0.5
RUNTIME_WEIGHT = 0.5
T_BLK = 512


def _stage1_body(h_ref, s_ref, w_ref, rsum_ref, prior_ref):
    h = h_ref[...]
    s = s_ref[...]
    d = h - s
    rsum_ref[...] = jnp.sum(d * d, axis=2)
    hb = h.astype(jnp.bfloat16).astype(jnp.float32)
    wb = w_ref[0].astype(jnp.bfloat16).astype(jnp.float32)[None, None, :]
    prior_ref[...] = jnp.sum(hb * wb, axis=2)


def _stage2_body(rsum_ref, praw_ref, scores_ref, sem_ref, pscore_ref,
                 rscore_ref, peak_ref):
    rraw = jnp.sqrt(rsum_ref[...])
    praw = praw_ref[...]

    def standardize(x):
        mean = jnp.mean(x, axis=1, keepdims=True)
        var = jnp.mean((x - mean) * (x - mean), axis=1, keepdims=True)
        std = jnp.maximum(jnp.sqrt(var), 1e-6)
        return (x - mean) / std

    runtime_logits = standardize(rraw)
    prior_logits = standardize(praw)
    combined = PRIOR_WEIGHT * prior_logits + RUNTIME_WEIGHT * runtime_logits
    scores = jax.nn.sigmoid(combined)
    scores_ref[...] = scores
    sem_ref[...] = combined
    pscore_ref[...] = jax.nn.sigmoid(prior_logits)
    rscore_ref[...] = jax.nn.sigmoid(runtime_logits)
    left = jnp.concatenate([scores[:, :1], scores[:, :-1]], axis=1)
    right = jnp.concatenate([scores[:, 1:], scores[:, -1:]], axis=1)
    peak_ref[...] = ((scores >= left) & (scores >= right)).astype(jnp.int8)


def kernel(hidden, history, W, b):
    d = hidden.shape[-1]
    n_t = T // T_BLK

    rsum, prior_raw = pl.pallas_call(
        _stage1_body,
        grid=(n_t,),
        in_specs=[
            pl.BlockSpec((B, T_BLK, D), lambda j: (0, j, 0)),
            pl.BlockSpec((B, T_BLK, D), lambda j: (0, j, 0)),
            pl.BlockSpec((1, D), lambda j: (0, 0)),
        ],
        out_specs=[
            pl.BlockSpec((B, T_BLK), lambda j: (0, j)),
            pl.BlockSpec((B, T_BLK), lambda j: (0, j)),
        ],
        out_shape=[
            jax.ShapeDtypeStruct((B, T), jnp.float32),
            jax.ShapeDtypeStruct((B, T), jnp.float32),
        ],
    )(hidden, history, W)

    del d, b

    scores, sem, pscore, rscore, peak_i8 = pl.pallas_call(
        _stage2_body,
        out_shape=[
            jax.ShapeDtypeStruct((B, T), jnp.float32),
            jax.ShapeDtypeStruct((B, T), jnp.float32),
            jax.ShapeDtypeStruct((B, T), jnp.float32),
            jax.ShapeDtypeStruct((B, T), jnp.float32),
            jax.ShapeDtypeStruct((B, T), jnp.int8),
        ],
    )(rsum, prior_raw)

    positions = jnp.arange(T, dtype=jnp.int32)
    starts = jnp.clip(positions - 1, 0, None)
    span_bounds = jnp.broadcast_to(
        jnp.stack((starts, positions), axis=-1)[None, :, :], (B, T, 2))

    return scores, span_bounds, sem, pscore, rscore, peak_i8.astype(jnp.bool_)


# speedup vs baseline: 1.4798x; 1.0161x over previous
import jax
import jax.numpy as jnp
from jax.experimental import pallas as pl
from jax.experimental.pallas import tpu as pltpu

B, T, D = 4, 8192, 1024
PRIOR_WEIGHT = 0.5
RUNTIME_WEIGHT = 0.5
T_BLK = 512
N_T = T // T_BLK


def _fused_body(h_ref, s_ref, w_ref, scores_ref, sem_ref, pscore_ref,
                rscore_ref, peak_ref, rsum_acc, prior_acc):
    j = pl.program_id(0)
    h = h_ref[...]
    s = s_ref[...]
    d = h - s
    rsum_acc[:, pl.ds(j * T_BLK, T_BLK)] = jnp.sum(d * d, axis=2)
    hb = h.astype(jnp.bfloat16).astype(jnp.float32)
    wb = w_ref[0].astype(jnp.bfloat16).astype(jnp.float32)[None, None, :]
    prior_acc[:, pl.ds(j * T_BLK, T_BLK)] = jnp.sum(hb * wb, axis=2)

    @pl.when(j == N_T - 1)
    def _finalize():
        rraw = jnp.sqrt(rsum_acc[...])
        praw = prior_acc[...]

        def standardize(x):
            mean = jnp.mean(x, axis=1, keepdims=True)
            var = jnp.mean((x - mean) * (x - mean), axis=1, keepdims=True)
            std = jnp.maximum(jnp.sqrt(var), 1e-6)
            return (x - mean) / std

        runtime_logits = standardize(rraw)
        prior_logits = standardize(praw)
        combined = (PRIOR_WEIGHT * prior_logits
                    + RUNTIME_WEIGHT * runtime_logits)
        scores = jax.nn.sigmoid(combined)
        scores_ref[...] = scores
        sem_ref[...] = combined
        pscore_ref[...] = jax.nn.sigmoid(prior_logits)
        rscore_ref[...] = jax.nn.sigmoid(runtime_logits)
        left = jnp.concatenate([scores[:, :1], scores[:, :-1]], axis=1)
        right = jnp.concatenate([scores[:, 1:], scores[:, -1:]], axis=1)
        peak_ref[...] = ((scores >= left) & (scores >= right)).astype(jnp.int8)


def kernel(hidden, history, W, b):
    del b

    full = pl.BlockSpec((B, T), lambda j: (0, 0))
    scores, sem, pscore, rscore, peak_i8 = pl.pallas_call(
        _fused_body,
        grid=(N_T,),
        in_specs=[
            pl.BlockSpec((B, T_BLK, D), lambda j: (0, j, 0)),
            pl.BlockSpec((B, T_BLK, D), lambda j: (0, j, 0)),
            pl.BlockSpec((1, D), lambda j: (0, 0)),
        ],
        out_specs=[full, full, full, full, full],
        out_shape=[
            jax.ShapeDtypeStruct((B, T), jnp.float32),
            jax.ShapeDtypeStruct((B, T), jnp.float32),
            jax.ShapeDtypeStruct((B, T), jnp.float32),
            jax.ShapeDtypeStruct((B, T), jnp.float32),
            jax.ShapeDtypeStruct((B, T), jnp.int8),
        ],
        scratch_shapes=[
            pltpu.VMEM((B, T), jnp.float32),
            pltpu.VMEM((B, T), jnp.float32),
        ],
    )(hidden, history, W)

    positions = jnp.arange(T, dtype=jnp.int32)
    starts = jnp.clip(positions - 1, 0, None)
    span_bounds = jnp.broadcast_to(
        jnp.stack((starts, positions), axis=-1)[None, :, :], (B, T, 2))

    return scores, span_bounds, sem, pscore, rscore, peak_i8.astype(jnp.bool_)


# speedup vs baseline: 1.4950x; 1.0103x over previous
import jax
import jax.numpy as jnp
from jax.experimental import pallas as pl
from jax.experimental.pallas import tpu as pltpu

B, T, D = 4, 8192, 1024
PRIOR_WEIGHT = 0.5
RUNTIME_WEIGHT = 0.5
T_BLK = 256
N_T = T // T_BLK


def _fused_body(h_ref, s_ref, w_ref, scores_ref, sem_ref, pscore_ref,
                rscore_ref, peak_ref, rsum_acc, prior_acc):
    j = pl.program_id(0)
    h = h_ref[...]
    s = s_ref[...]
    d = h - s
    rsum_acc[:, pl.ds(j * T_BLK, T_BLK)] = jnp.sum(d * d, axis=2)
    hb = h.astype(jnp.bfloat16).astype(jnp.float32)
    wb = w_ref[0].astype(jnp.bfloat16).astype(jnp.float32)[None, None, :]
    prior_acc[:, pl.ds(j * T_BLK, T_BLK)] = jnp.sum(hb * wb, axis=2)

    @pl.when(j == N_T - 1)
    def _finalize():
        rraw = jnp.sqrt(rsum_acc[...])
        praw = prior_acc[...]

        def standardize(x):
            mean = jnp.mean(x, axis=1, keepdims=True)
            var = jnp.mean((x - mean) * (x - mean), axis=1, keepdims=True)
            std = jnp.maximum(jnp.sqrt(var), 1e-6)
            return (x - mean) / std

        runtime_logits = standardize(rraw)
        prior_logits = standardize(praw)
        combined = (PRIOR_WEIGHT * prior_logits
                    + RUNTIME_WEIGHT * runtime_logits)
        scores = jax.nn.sigmoid(combined)
        scores_ref[...] = scores
        sem_ref[...] = combined
        pscore_ref[...] = jax.nn.sigmoid(prior_logits)
        rscore_ref[...] = jax.nn.sigmoid(runtime_logits)
        left = jnp.concatenate([scores[:, :1], scores[:, :-1]], axis=1)
        right = jnp.concatenate([scores[:, 1:], scores[:, -1:]], axis=1)
        peak_ref[...] = ((scores >= left) & (scores >= right)).astype(jnp.int8)


def kernel(hidden, history, W, b):
    del b

    full = pl.BlockSpec((B, T), lambda j: (0, 0))
    scores, sem, pscore, rscore, peak_i8 = pl.pallas_call(
        _fused_body,
        grid=(N_T,),
        in_specs=[
            pl.BlockSpec((B, T_BLK, D), lambda j: (0, j, 0)),
            pl.BlockSpec((B, T_BLK, D), lambda j: (0, j, 0)),
            pl.BlockSpec((1, D), lambda j: (0, 0)),
        ],
        out_specs=[full, full, full, full, full],
        out_shape=[
            jax.ShapeDtypeStruct((B, T), jnp.float32),
            jax.ShapeDtypeStruct((B, T), jnp.float32),
            jax.ShapeDtypeStruct((B, T), jnp.float32),
            jax.ShapeDtypeStruct((B, T), jnp.float32),
            jax.ShapeDtypeStruct((B, T), jnp.int8),
        ],
        scratch_shapes=[
            pltpu.VMEM((B, T), jnp.float32),
            pltpu.VMEM((B, T), jnp.float32),
        ],
    )(hidden, history, W)

    positions = jnp.arange(T, dtype=jnp.int32)
    starts = jnp.clip(positions - 1, 0, None)
    span_bounds = jnp.broadcast_to(
        jnp.stack((starts, positions), axis=-1)[None, :, :], (B, T, 2))

    return scores, span_bounds, sem, pscore, rscore, peak_i8.astype(jnp.bool_)
